# baseline (speedup 1.0000x reference)
import functools

import jax
import jax.numpy as jnp
from jax import lax
from jax.experimental import pallas as pl
from jax.experimental.pallas import tpu as pltpu
from jax.experimental.pallas import tpu_sc as plsc

N_SAMPLE = 100000
N_LATENT = 16
N_OUT = 16
BATCH = 16384

NC = 2
NS = 16
NW = NC * NS
ROWS_PER_W = BATCH // NW
CHUNK = 128
N_CHUNKS = ROWS_PER_W // CHUNK


def _sc_body(u_hbm, idx_hbm, amat_hbm, offs_hbm, out_hbm,
             idx_v0, idx_v1, a_v0, a_v1, off_v0, off_v1, u_v, out_v,
             sem_a0, sem_a1, sem_o0, sem_o1):
    wid = lax.axis_index("s") * NC + lax.axis_index("c")
    base = wid * ROWS_PER_W

    idx_b = [idx_v0, idx_v1]
    a_b = [a_v0, a_v1]
    off_b = [off_v0, off_v1]
    sem_a = [sem_a0, sem_a1]
    sem_o = [sem_o0, sem_o1]

    lane_ids = [jnp.full((16,), l, dtype=jnp.int32) for l in range(N_LATENT)]

    def issue(c, b):
        cbase = base + c * CHUNK
        pltpu.sync_copy(idx_hbm.at[pl.ds(cbase, CHUNK)], idx_b[b])
        cp_a = pltpu.async_copy(amat_hbm.at[idx_b[b]], a_b[b], sem_a[b])
        cp_o = pltpu.async_copy(offs_hbm.at[idx_b[b]], off_b[b], sem_o[b])
        return cp_a, cp_o

    cps = issue(0, 0)
    for c in range(N_CHUNKS):
        b = c % 2
        nxt = issue(c + 1, 1 - b) if c + 1 < N_CHUNKS else None
        cbase = base + c * CHUNK
        pltpu.sync_copy(u_hbm.at[pl.ds(cbase * 16, CHUNK * 16)], u_v)
        cps[0].wait()
        cps[1].wait()
        a_v = a_b[b]
        off_v = off_b[b]

        def row_body(i, carry):
            for rr in range(4):
                r = lax.add(lax.mul(i, 4), rr)
                uvec = u_v[pl.ds(r * 16, 16)]
                off_r = off_v[r, pl.ds(0, 16)]
                accs = [uvec + off_r, None, None, None]
                for q in range(4):
                    for l in range(q * 4, q * 4 + 4):
                        a_l = a_v[r, pl.ds(l * 16, 16)]
                        u_l = uvec.at[lane_ids[l]].get(
                            mode="promise_in_bounds")
                        t = u_l * a_l
                        accs[q] = t if accs[q] is None else accs[q] + t
                acc = (accs[0] + accs[1]) + (accs[2] + accs[3])
                out_v[pl.ds(r * 16, 16)] = acc
            return carry

        lax.fori_loop(0, CHUNK // 4, row_body, 0)
        pltpu.sync_copy(out_v, out_hbm.at[pl.ds(cbase * 16, CHUNK * 16)])
        cps = nxt


@jax.jit
def kernel(u, sample_index, amat_sample, offsets):
    idx = jnp.squeeze(sample_index).astype(jnp.int32)
    amat2d = amat_sample.reshape(N_SAMPLE, N_LATENT * N_OUT)
    u_flat = u.reshape(BATCH * N_LATENT)

    mesh = plsc.VectorSubcoreMesh(
        core_axis_name="c", subcore_axis_name="s",
        num_cores=NC, num_subcores=NS)
    run = pl.kernel(
        _sc_body,
        out_type=jax.ShapeDtypeStruct((BATCH * N_OUT,), jnp.float32),
        mesh=mesh,
        scratch_types=[
            pltpu.VMEM((CHUNK,), jnp.int32),
            pltpu.VMEM((CHUNK,), jnp.int32),
            pltpu.VMEM((CHUNK, N_LATENT * N_OUT), jnp.float32),
            pltpu.VMEM((CHUNK, N_LATENT * N_OUT), jnp.float32),
            pltpu.VMEM((CHUNK, N_OUT), jnp.float32),
            pltpu.VMEM((CHUNK, N_OUT), jnp.float32),
            pltpu.VMEM((CHUNK * N_LATENT,), jnp.float32),
            pltpu.VMEM((CHUNK * N_OUT,), jnp.float32),
            pltpu.SemaphoreType.DMA,
            pltpu.SemaphoreType.DMA,
            pltpu.SemaphoreType.DMA,
            pltpu.SemaphoreType.DMA,
        ],
        compiler_params=pltpu.CompilerParams(
            needs_layout_passes=False, use_tc_tiling_on_sc=False),
    )
    out_flat = run(u_flat, idx, amat2d, offsets)
    return out_flat.reshape(BATCH, N_OUT)


# speedup vs baseline: 1.1955x; 1.1955x over previous
import functools

import jax
import jax.numpy as jnp
from jax import lax
from jax.experimental import pallas as pl
from jax.experimental.pallas import tpu as pltpu
from jax.experimental.pallas import tpu_sc as plsc

N_SAMPLE = 100000
N_LATENT = 16
N_OUT = 16
BATCH = 16384

NC = 2
NS = 16
NW = NC * NS
ROWS_PER_W = BATCH // NW
CHUNK = 128
N_CHUNKS = ROWS_PER_W // CHUNK
NSTREAM = 8
SUB = CHUNK // NSTREAM


def _sc_body(u_hbm, idx_hbm, amat_hbm, offs_hbm, out_hbm,
             idx_v0, idx_v1, idx8_v0, idx8_v1, cs_v0, cs_v1,
             a_v0, a_v1, off_v0, off_v1, u_v, out_v,
             sem_a0, sem_a1, sem_o0, sem_o1):
    wid = lax.axis_index("s") * NC + lax.axis_index("c")
    base = wid * ROWS_PER_W

    idx_b = [idx_v0, idx_v1]
    idx8_b = [idx8_v0, idx8_v1]
    cs_b = [cs_v0, cs_v1]
    a_b = [a_v0, a_v1]
    off_b = [off_v0, off_v1]
    sem_a = [sem_a0, sem_a1]
    sem_o = [sem_o0, sem_o1]

    lane_ids = [jnp.full((16,), l, dtype=jnp.int32) for l in range(N_LATENT)]
    iota16 = lax.iota(jnp.int32, 16)

    def issue(c, b):
        cbase = base + c * CHUNK
        pltpu.sync_copy(idx_hbm.at[pl.ds(cbase, CHUNK)], idx_b[b])
        for g in range(CHUNK // 16):
            iv = idx_b[b][pl.ds(g * 16, 16)]
            idx8_b[b][pl.ds(g * 16, 16)] = lax.shift_right_logical(iv, 3)
            cs_b[b][pl.ds(g * 16, 16)] = lax.shift_left(
                lax.bitwise_and(iv, jnp.int32(7)), 4)
        cps = []
        for s in range(NSTREAM):
            cps.append(pltpu.async_copy(
                amat_hbm.at[idx_b[b].at[pl.ds(s * SUB, SUB)]],
                a_b[b].at[pl.ds(s * SUB, SUB)],
                sem_a[b]))
        cps.append(pltpu.async_copy(
            offs_hbm.at[idx8_b[b]], off_b[b], sem_o[b]))
        return cps

    cps = issue(0, 0)
    for c in range(N_CHUNKS):
        b = c % 2
        nxt = issue(c + 1, 1 - b) if c + 1 < N_CHUNKS else None
        cbase = base + c * CHUNK
        pltpu.sync_copy(u_hbm.at[pl.ds(cbase * 16, CHUNK * 16)], u_v)
        for cp in cps:
            cp.wait()
        a_v = a_b[b]
        off_v = off_b[b]
        cs_v = cs_b[b]

        def row_body(i, carry):
            for rr in range(4):
                r = lax.add(lax.mul(i, 4), rr)
                uvec = u_v[pl.ds(r * 16, 16)]
                g0 = lax.mul(lax.div(r, 16), 16)
                cs_grp = cs_v[pl.ds(g0, 16)]
                cs = cs_grp.at[jnp.full((16,), lax.rem(r, 16), jnp.int32)].get(
                    mode="promise_in_bounds")
                off_r = plsc.load_gather(
                    off_v, [jnp.full((16,), r, jnp.int32), cs + iota16])
                accs = [uvec + off_r, None, None, None]
                for q in range(4):
                    for l in range(q * 4, q * 4 + 4):
                        a_l = a_v[r, pl.ds(l * 16, 16)]
                        u_l = uvec.at[lane_ids[l]].get(
                            mode="promise_in_bounds")
                        t = u_l * a_l
                        accs[q] = t if accs[q] is None else accs[q] + t
                acc = (accs[0] + accs[1]) + (accs[2] + accs[3])
                out_v[pl.ds(r * 16, 16)] = acc
            return carry

        lax.fori_loop(0, CHUNK // 4, row_body, 0)
        pltpu.sync_copy(out_v, out_hbm.at[pl.ds(cbase * 16, CHUNK * 16)])
        cps = nxt


@jax.jit
def kernel(u, sample_index, amat_sample, offsets):
    idx = jnp.squeeze(sample_index).astype(jnp.int32)
    amat2d = amat_sample.reshape(N_SAMPLE, N_LATENT * N_OUT)
    offs128 = offsets.reshape(N_SAMPLE // 8, 128)
    u_flat = u.reshape(BATCH * N_LATENT)

    mesh = plsc.VectorSubcoreMesh(
        core_axis_name="c", subcore_axis_name="s",
        num_cores=NC, num_subcores=NS)
    run = pl.kernel(
        _sc_body,
        out_type=jax.ShapeDtypeStruct((BATCH * N_OUT,), jnp.float32),
        mesh=mesh,
        scratch_types=[
            pltpu.VMEM((CHUNK,), jnp.int32),
            pltpu.VMEM((CHUNK,), jnp.int32),
            pltpu.VMEM((CHUNK,), jnp.int32),
            pltpu.VMEM((CHUNK,), jnp.int32),
            pltpu.VMEM((CHUNK,), jnp.int32),
            pltpu.VMEM((CHUNK,), jnp.int32),
            pltpu.VMEM((CHUNK, N_LATENT * N_OUT), jnp.float32),
            pltpu.VMEM((CHUNK, N_LATENT * N_OUT), jnp.float32),
            pltpu.VMEM((CHUNK, 128), jnp.float32),
            pltpu.VMEM((CHUNK, 128), jnp.float32),
            pltpu.VMEM((CHUNK * N_LATENT,), jnp.float32),
            pltpu.VMEM((CHUNK * N_OUT,), jnp.float32),
            pltpu.SemaphoreType.DMA,
            pltpu.SemaphoreType.DMA,
            pltpu.SemaphoreType.DMA,
            pltpu.SemaphoreType.DMA,
        ],
        compiler_params=pltpu.CompilerParams(needs_layout_passes=False),
    )
    out_flat = run(u_flat, idx, amat2d, offs128)
    return out_flat.reshape(BATCH, N_OUT)


# speedup vs baseline: 1.1994x; 1.0033x over previous
import functools

import jax
import jax.numpy as jnp
from jax import lax
from jax.experimental import pallas as pl
from jax.experimental.pallas import tpu as pltpu
from jax.experimental.pallas import tpu_sc as plsc

N_SAMPLE = 100000
N_LATENT = 16
N_OUT = 16
BATCH = 16384

NC = 2
NS = 16
NW = NC * NS
ROWS_PER_W = BATCH // NW
CHUNK = 128
N_CHUNKS = ROWS_PER_W // CHUNK


def _sc_body(u_hbm, idx_hbm, amat_hbm, offs_hbm, out_hbm,
             idx_v0, idx_v1, idx8_v0, idx8_v1, cs_v0, cs_v1,
             a_v0, a_v1, off_v0, off_v1, u_v, out_v,
             sem_a0, sem_a1, sem_o0, sem_o1):
    wid = lax.axis_index("s") * NC + lax.axis_index("c")
    base = wid * ROWS_PER_W

    idx_b = [idx_v0, idx_v1]
    idx8_b = [idx8_v0, idx8_v1]
    cs_b = [cs_v0, cs_v1]
    a_b = [a_v0, a_v1]
    off_b = [off_v0, off_v1]
    sem_a = [sem_a0, sem_a1]
    sem_o = [sem_o0, sem_o1]

    lane_ids = [jnp.full((16,), l, dtype=jnp.int32) for l in range(N_LATENT)]
    iota16 = lax.iota(jnp.int32, 16)

    def issue(c, b):
        cbase = base + c * CHUNK
        pltpu.sync_copy(idx_hbm.at[pl.ds(cbase, CHUNK)], idx_b[b])
        for g in range(CHUNK // 16):
            iv = idx_b[b][pl.ds(g * 16, 16)]
            idx8_b[b][pl.ds(g * 16, 16)] = lax.shift_right_logical(iv, 3)
            cs_b[b][pl.ds(g * 16, 16)] = lax.shift_left(
                lax.bitwise_and(iv, jnp.int32(7)), 4)
        cp_a = pltpu.async_copy(amat_hbm.at[idx_b[b]], a_b[b], sem_a[b])
        cp_o = pltpu.async_copy(offs_hbm.at[idx8_b[b]], off_b[b], sem_o[b])
        return cp_a, cp_o

    cps = issue(0, 0)
    for c in range(N_CHUNKS):
        b = c % 2
        nxt = issue(c + 1, 1 - b) if c + 1 < N_CHUNKS else None
        cbase = base + c * CHUNK
        pltpu.sync_copy(u_hbm.at[pl.ds(cbase * 16, CHUNK * 16)], u_v)
        cps[0].wait()
        cps[1].wait()
        a_v = a_b[b]
        off_v = off_b[b]
        cs_v = cs_b[b]

        def row_body(i, carry):
            for rr in range(4):
                r = lax.add(lax.mul(i, 4), rr)
                uvec = u_v[pl.ds(r * 16, 16)]
                g0 = lax.mul(lax.div(r, 16), 16)
                cs_grp = cs_v[pl.ds(g0, 16)]
                cs = cs_grp.at[jnp.full((16,), lax.rem(r, 16), jnp.int32)].get(
                    mode="promise_in_bounds")
                off_r = plsc.load_gather(
                    off_v, [jnp.full((16,), r, jnp.int32), cs + iota16])
                accs = [uvec + off_r, None, None, None]
                for q in range(4):
                    for l in range(q * 4, q * 4 + 4):
                        a_l = a_v[r, pl.ds(l * 16, 16)]
                        u_l = uvec.at[lane_ids[l]].get(
                            mode="promise_in_bounds")
                        t = u_l * a_l
                        accs[q] = t if accs[q] is None else accs[q] + t
                acc = (accs[0] + accs[1]) + (accs[2] + accs[3])
                out_v[pl.ds(r * 16, 16)] = acc
            return carry

        lax.fori_loop(0, CHUNK // 4, row_body, 0)
        pltpu.sync_copy(out_v, out_hbm.at[pl.ds(cbase * 16, CHUNK * 16)])
        cps = nxt


@jax.jit
def kernel(u, sample_index, amat_sample, offsets):
    idx = jnp.squeeze(sample_index).astype(jnp.int32)
    amat2d = amat_sample.reshape(N_SAMPLE, N_LATENT * N_OUT)
    offs128 = offsets.reshape(N_SAMPLE // 8, 128)
    u_flat = u.reshape(BATCH * N_LATENT)

    mesh = plsc.VectorSubcoreMesh(
        core_axis_name="c", subcore_axis_name="s",
        num_cores=NC, num_subcores=NS)
    run = pl.kernel(
        _sc_body,
        out_type=jax.ShapeDtypeStruct((BATCH * N_OUT,), jnp.float32),
        mesh=mesh,
        scratch_types=[
            pltpu.VMEM((CHUNK,), jnp.int32),
            pltpu.VMEM((CHUNK,), jnp.int32),
            pltpu.VMEM((CHUNK,), jnp.int32),
            pltpu.VMEM((CHUNK,), jnp.int32),
            pltpu.VMEM((CHUNK,), jnp.int32),
            pltpu.VMEM((CHUNK,), jnp.int32),
            pltpu.VMEM((CHUNK, N_LATENT * N_OUT), jnp.float32),
            pltpu.VMEM((CHUNK, N_LATENT * N_OUT), jnp.float32),
            pltpu.VMEM((CHUNK, 128), jnp.float32),
            pltpu.VMEM((CHUNK, 128), jnp.float32),
            pltpu.VMEM((CHUNK * N_LATENT,), jnp.float32),
            pltpu.VMEM((CHUNK * N_OUT,), jnp.float32),
            pltpu.SemaphoreType.DMA,
            pltpu.SemaphoreType.DMA,
            pltpu.SemaphoreType.DMA,
            pltpu.SemaphoreType.DMA,
        ],
        compiler_params=pltpu.CompilerParams(needs_layout_passes=False),
    )
    out_flat = run(u_flat, idx, amat2d, offs128)
    return out_flat.reshape(BATCH, N_OUT)


def _debug_entry_layout():
    import sys
    try:
        shapes = (
            jax.ShapeDtypeStruct((BATCH, N_LATENT), jnp.float32),
            jax.ShapeDtypeStruct((BATCH,), jnp.int32),
            jax.ShapeDtypeStruct((N_SAMPLE, N_LATENT, N_OUT), jnp.float32),
            jax.ShapeDtypeStruct((N_SAMPLE, N_OUT), jnp.float32),
        )
        txt = jax.jit(kernel).lower(*shapes).compile().as_text()
        for ln in txt.splitlines():
            if "entry_computation_layout" in ln:
                print("DBGFMT entry", ln.strip()[:600], file=sys.stderr)
                break
        for ln in txt.splitlines():
            ls = ln.strip()
            if ls.startswith(("%copy", "ROOT %copy")) or "fusion" in ls[:40]:
                print("DBGFMT", ls[:180], file=sys.stderr)
    except Exception as e:
        print("DBGFMT err", repr(e)[:300], file=sys.stderr)


_debug_entry_layout()
